# speedup vs baseline: 166.2605x; 166.2605x over previous
import functools

import jax
import jax.numpy as jnp
import numpy as np
from jax.experimental import pallas as pl
from jax.experimental.pallas import tpu as pltpu

ETA_MU = 0.01
ETA_VAR = 0.02

BD = 512
CO = 512
CS = 128
TCS = 1024


def _stats_kernel(nt, x_ref, o_ref, acc_ref):
    t = pl.program_id(1)

    @pl.when(t == 0)
    def _():
        acc_ref[...] = jnp.zeros_like(acc_ref)

    xb = x_ref[...]
    acc_ref[0:1, :] += jnp.sum(xb, axis=0, keepdims=True)
    acc_ref[1:2, :] += jnp.sum(xb * xb, axis=0, keepdims=True)

    @pl.when(t == nt - 1)
    def _():
        total = jnp.float32(nt * TCS)
        s1 = acc_ref[0:1, :]
        s2 = acc_ref[1:2, :]
        mean = s1 / total
        var = (s2 - s1 * mean) / (total - 1.0)
        o_ref[0:1, :] = mean
        o_ref[1:2, :] = jnp.sqrt(jnp.maximum(var, 0.0))


def _scan_kernel(nt, x_ref, ms_ref, lmu_ref, lv_ref, pmu_ref, pv_ref,
                 norm_ref, info_ref, mu_c, var_c):
    t = pl.program_id(1)

    @pl.when(t == 0)
    def _():
        mu_c[...] = ms_ref[0:1, :]
        var_c[...] = ms_ref[1:2, :]

    mu_carry = mu_c[...]
    var_carry = var_c[...]
    lmu = lmu_ref[...]
    lv = lv_ref[...]
    pmu = pmu_ref[...]
    pv = pv_ref[...]
    for s in range(CO // CS):
        sl = slice(s * CS, (s + 1) * CS)
        xs = x_ref[sl, :]
        mu = jnp.dot(lmu, xs, preferred_element_type=jnp.float32)
        mu += pmu * mu_carry
        diff = xs - mu
        e = diff * diff
        var = jnp.dot(lv, e, preferred_element_type=jnp.float32)
        var += pv * var_carry
        norm_ref[sl, :] = diff * jax.lax.rsqrt(var)
        info_ref[sl, 0, :] = mu
        info_ref[sl, 1, :] = var
        mu_carry = mu[CS - 1:CS, :]
        var_carry = var[CS - 1:CS, :]
    mu_c[...] = mu_carry
    var_c[...] = var_carry


def _tri_mats():
    i = np.arange(CS)
    diff = i[:, None] - i[None, :]
    a = 1.0 - ETA_MU
    c = 1.0 - ETA_VAR
    lmu = np.where(diff >= 0, ETA_MU * a ** np.maximum(diff, 0), 0.0)
    lv = np.where(diff >= 0, ETA_VAR * c ** np.maximum(diff, 0), 0.0)
    pmu = a ** (i + 1.0)
    pv = c ** (i + 1.0)
    return (jnp.asarray(lmu, jnp.float32), jnp.asarray(lv, jnp.float32),
            jnp.asarray(pmu[:, None], jnp.float32),
            jnp.asarray(pv[:, None], jnp.float32))


def kernel(x):
    T, D = x.shape
    nd = D // BD

    nts = T // TCS
    ms = pl.pallas_call(
        functools.partial(_stats_kernel, nts),
        out_shape=jax.ShapeDtypeStruct((2, D), jnp.float32),
        grid=(nd, nts),
        in_specs=[pl.BlockSpec((TCS, BD), lambda d, t: (t, d))],
        out_specs=pl.BlockSpec((2, BD), lambda d, t: (0, d)),
        scratch_shapes=[pltpu.VMEM((2, BD), jnp.float32)],
        compiler_params=pltpu.CompilerParams(
            dimension_semantics=("parallel", "arbitrary")),
        name="gas_stats",
    )(x)

    lmu, lv, pmu, pv = _tri_mats()
    nt = T // CO
    norm, info = pl.pallas_call(
        functools.partial(_scan_kernel, nt),
        out_shape=(jax.ShapeDtypeStruct((T, D), jnp.float32),
                   jax.ShapeDtypeStruct((T, 2, D), jnp.float32)),
        grid=(nd, nt),
        in_specs=[
            pl.BlockSpec((CO, BD), lambda d, t: (t, d)),
            pl.BlockSpec((2, BD), lambda d, t: (0, d)),
            pl.BlockSpec((CS, CS), lambda d, t: (0, 0)),
            pl.BlockSpec((CS, CS), lambda d, t: (0, 0)),
            pl.BlockSpec((CS, 1), lambda d, t: (0, 0)),
            pl.BlockSpec((CS, 1), lambda d, t: (0, 0)),
        ],
        out_specs=(pl.BlockSpec((CO, BD), lambda d, t: (t, d)),
                   pl.BlockSpec((CO, 2, BD), lambda d, t: (t, 0, d))),
        scratch_shapes=[pltpu.VMEM((1, BD), jnp.float32),
                        pltpu.VMEM((1, BD), jnp.float32)],
        compiler_params=pltpu.CompilerParams(
            dimension_semantics=("parallel", "arbitrary")),
        name="gas_scan",
    )(x, ms, lmu, lv, pmu, pv)

    return norm, info.reshape(T, 2 * D)


# speedup vs baseline: 169.1667x; 1.0175x over previous
import functools

import jax
import jax.numpy as jnp
import numpy as np
from jax.experimental import pallas as pl
from jax.experimental.pallas import tpu as pltpu

ETA_MU = 0.01
ETA_VAR = 0.02

BD = 512
CO = 512
CS = 256
TCS = 1024


def _stats_kernel(nt, x_ref, o_ref, acc_ref):
    t = pl.program_id(1)

    @pl.when(t == 0)
    def _():
        acc_ref[...] = jnp.zeros_like(acc_ref)

    xb = x_ref[...]
    acc_ref[0:1, :] += jnp.sum(xb, axis=0, keepdims=True)
    acc_ref[1:2, :] += jnp.sum(xb * xb, axis=0, keepdims=True)

    @pl.when(t == nt - 1)
    def _():
        total = jnp.float32(nt * TCS)
        s1 = acc_ref[0:1, :]
        s2 = acc_ref[1:2, :]
        mean = s1 / total
        var = (s2 - s1 * mean) / (total - 1.0)
        o_ref[0:1, :] = mean
        o_ref[1:2, :] = jnp.sqrt(jnp.maximum(var, 0.0))


def _scan_kernel(nt, x_ref, ms_ref, lmu_ref, lv_ref, pmu_ref, pv_ref,
                 norm_ref, info_ref, mu_c, var_c):
    t = pl.program_id(1)

    @pl.when(t == 0)
    def _():
        mu_c[...] = ms_ref[0:1, :]
        var_c[...] = ms_ref[1:2, :]

    mu_carry = mu_c[...]
    var_carry = var_c[...]
    lmu = lmu_ref[...]
    lv = lv_ref[...]
    pmu = pmu_ref[...]
    pv = pv_ref[...]
    for s in range(CO // CS):
        sl = slice(s * CS, (s + 1) * CS)
        xs = x_ref[sl, :]
        mu = jnp.dot(lmu, xs, preferred_element_type=jnp.float32)
        mu += pmu * mu_carry
        diff = xs - mu
        e = diff * diff
        var = jnp.dot(lv, e, preferred_element_type=jnp.float32)
        var += pv * var_carry
        norm_ref[sl, :] = diff * jax.lax.rsqrt(var)
        info_ref[sl, 0, :] = mu
        info_ref[sl, 1, :] = var
        mu_carry = mu[CS - 1:CS, :]
        var_carry = var[CS - 1:CS, :]
    mu_c[...] = mu_carry
    var_c[...] = var_carry


def _tri_mats():
    i = np.arange(CS)
    diff = i[:, None] - i[None, :]
    a = 1.0 - ETA_MU
    c = 1.0 - ETA_VAR
    lmu = np.where(diff >= 0, ETA_MU * a ** np.maximum(diff, 0), 0.0)
    lv = np.where(diff >= 0, ETA_VAR * c ** np.maximum(diff, 0), 0.0)
    pmu = a ** (i + 1.0)
    pv = c ** (i + 1.0)
    return (jnp.asarray(lmu, jnp.float32), jnp.asarray(lv, jnp.float32),
            jnp.asarray(pmu[:, None], jnp.float32),
            jnp.asarray(pv[:, None], jnp.float32))


def kernel(x):
    T, D = x.shape
    nd = D // BD

    nts = T // TCS
    ms = pl.pallas_call(
        functools.partial(_stats_kernel, nts),
        out_shape=jax.ShapeDtypeStruct((2, D), jnp.float32),
        grid=(nd, nts),
        in_specs=[pl.BlockSpec((TCS, BD), lambda d, t: (t, d))],
        out_specs=pl.BlockSpec((2, BD), lambda d, t: (0, d)),
        scratch_shapes=[pltpu.VMEM((2, BD), jnp.float32)],
        compiler_params=pltpu.CompilerParams(
            dimension_semantics=("parallel", "arbitrary")),
        name="gas_stats",
    )(x)

    lmu, lv, pmu, pv = _tri_mats()
    nt = T // CO
    norm, info = pl.pallas_call(
        functools.partial(_scan_kernel, nt),
        out_shape=(jax.ShapeDtypeStruct((T, D), jnp.float32),
                   jax.ShapeDtypeStruct((T, 2, D), jnp.float32)),
        grid=(nd, nt),
        in_specs=[
            pl.BlockSpec((CO, BD), lambda d, t: (t, d)),
            pl.BlockSpec((2, BD), lambda d, t: (0, d)),
            pl.BlockSpec((CS, CS), lambda d, t: (0, 0)),
            pl.BlockSpec((CS, CS), lambda d, t: (0, 0)),
            pl.BlockSpec((CS, 1), lambda d, t: (0, 0)),
            pl.BlockSpec((CS, 1), lambda d, t: (0, 0)),
        ],
        out_specs=(pl.BlockSpec((CO, BD), lambda d, t: (t, d)),
                   pl.BlockSpec((CO, 2, BD), lambda d, t: (t, 0, d))),
        scratch_shapes=[pltpu.VMEM((1, BD), jnp.float32),
                        pltpu.VMEM((1, BD), jnp.float32)],
        compiler_params=pltpu.CompilerParams(
            dimension_semantics=("parallel", "arbitrary")),
        name="gas_scan",
    )(x, ms, lmu, lv, pmu, pv)

    return norm, info.reshape(T, 2 * D)


# speedup vs baseline: 344.3745x; 2.0357x over previous
import functools

import jax
import jax.numpy as jnp
import numpy as np
from jax.experimental import pallas as pl
from jax.experimental.pallas import tpu as pltpu

ETA_MU = 0.01
ETA_VAR = 0.02

BD = 512
CO = 512
CS = 256
TCS = 1024
NC = 2
NW = 3


def _stats_kernel(nt, x_ref, o_ref, acc_ref):
    t = pl.program_id(1)

    @pl.when(t == 0)
    def _():
        acc_ref[...] = jnp.zeros_like(acc_ref)

    xb = x_ref[...]
    acc_ref[0:1, :] += jnp.sum(xb, axis=0, keepdims=True)
    acc_ref[1:2, :] += jnp.sum(xb * xb, axis=0, keepdims=True)

    @pl.when(t == nt - 1)
    def _():
        total = jnp.float32(nt * TCS)
        s1 = acc_ref[0:1, :]
        s2 = acc_ref[1:2, :]
        mean = s1 / total
        var = (s2 - s1 * mean) / (total - 1.0)
        o_ref[0:1, :] = mean
        o_ref[1:2, :] = jnp.sqrt(jnp.maximum(var, 0.0))


def _scan_kernel(nw, d_cols, x_ref, ms_ref, lmu_ref, lv_ref, pmu_ref, pv_ref,
                 norm_ref, info_ref, mu_c, var_c):
    h = pl.program_id(0)
    j = pl.program_id(1)

    @pl.when(jnp.logical_or(j == 0, jnp.logical_and(h == 0, j == nw)))
    def _():
        mu_c[...] = ms_ref[0:1, :]
        var_c[...] = ms_ref[1:2, :]

    mu_carry = mu_c[...]
    var_carry = var_c[...]
    lmu = lmu_ref[...]
    lv = lv_ref[...]
    pmu = pmu_ref[...]
    pv = pv_ref[...]
    for s in range(CO // CS):
        sl = slice(s * CS, (s + 1) * CS)
        xs = x_ref[sl, :]
        mu = jnp.dot(lmu, xs, preferred_element_type=jnp.float32)
        mu += pmu * mu_carry
        diff = xs - mu
        e = diff * diff
        var = jnp.dot(lv, e, preferred_element_type=jnp.float32)
        var += pv * var_carry
        norm_ref[sl, :] = diff * jax.lax.rsqrt(var)
        info_ref[sl, 0:d_cols] = mu
        info_ref[sl, d_cols:2 * d_cols] = var
        mu_carry = mu[CS - 1:CS, :]
        var_carry = var[CS - 1:CS, :]
    mu_c[...] = mu_carry
    var_c[...] = var_carry


def _tri_mats():
    i = np.arange(CS)
    diff = i[:, None] - i[None, :]
    a = 1.0 - ETA_MU
    c = 1.0 - ETA_VAR
    lmu = np.where(diff >= 0, ETA_MU * a ** np.maximum(diff, 0), 0.0)
    lv = np.where(diff >= 0, ETA_VAR * c ** np.maximum(diff, 0), 0.0)
    pmu = a ** (i + 1.0)
    pv = c ** (i + 1.0)
    return (jnp.asarray(lmu, jnp.float32), jnp.asarray(lv, jnp.float32),
            jnp.asarray(pmu[:, None], jnp.float32),
            jnp.asarray(pv[:, None], jnp.float32))


def kernel(x):
    T, D = x.shape
    nd = D // BD

    nts = T // TCS
    ms = pl.pallas_call(
        functools.partial(_stats_kernel, nts),
        out_shape=jax.ShapeDtypeStruct((2, D), jnp.float32),
        grid=(nd, nts),
        in_specs=[pl.BlockSpec((TCS, BD), lambda d, t: (t, d))],
        out_specs=pl.BlockSpec((2, BD), lambda d, t: (0, d)),
        scratch_shapes=[pltpu.VMEM((2, BD), jnp.float32)],
        compiler_params=pltpu.CompilerParams(
            dimension_semantics=("parallel", "arbitrary")),
        name="gas_stats",
    )(x)

    lmu, lv, pmu, pv = _tri_mats()
    nt_half = T // (NC * CO)
    nj = nt_half + NW

    def in_map(h, j):
        return (jnp.maximum(h * nt_half + j - NW, 0), 0)

    def out_map(h, j):
        return (h * nt_half + jnp.maximum(j - NW, 0), 0)

    norm, info = pl.pallas_call(
        functools.partial(_scan_kernel, NW, D),
        out_shape=(jax.ShapeDtypeStruct((T, D), jnp.float32),
                   jax.ShapeDtypeStruct((T, 2 * D), jnp.float32)),
        grid=(NC, nj),
        in_specs=[
            pl.BlockSpec((CO, D), in_map),
            pl.BlockSpec((2, D), lambda h, j: (0, 0)),
            pl.BlockSpec((CS, CS), lambda h, j: (0, 0)),
            pl.BlockSpec((CS, CS), lambda h, j: (0, 0)),
            pl.BlockSpec((CS, 1), lambda h, j: (0, 0)),
            pl.BlockSpec((CS, 1), lambda h, j: (0, 0)),
        ],
        out_specs=(pl.BlockSpec((CO, D), out_map),
                   pl.BlockSpec((CO, 2 * D), out_map)),
        scratch_shapes=[pltpu.VMEM((1, D), jnp.float32),
                        pltpu.VMEM((1, D), jnp.float32)],
        compiler_params=pltpu.CompilerParams(
            dimension_semantics=("parallel", "arbitrary")),
        name="gas_scan",
    )(x, ms, lmu, lv, pmu, pv)

    return norm, info
